# baseline (speedup 1.0000x reference)
import functools

import jax
import jax.numpy as jnp
from jax import lax
from jax.experimental import pallas as pl
from jax.experimental.pallas import tpu as pltpu
from jax.experimental.pallas import tpu_sc as plsc

N = 10000
D = 128
G = 64
NC = 2
NS = 16
NW = NC * NS
CHUNK = 128
KT = 160
EPAD = NS * KT * CHUNK
NBUF = 4
NSEG = 4
SEG = KT // NSEG
NPAD = 10240
HALF = NPAD // 2
ACCR = 5248
DUMMY = HALF
RPTA = ACCR // NS
BR = 1024
GRID = NPAD // BR


def _sc_mesh():
    return plsc.VectorSubcoreMesh(core_axis_name="c", subcore_axis_name="s")


def _sc_degree(dst2d, onesrow, zerosN):
    @functools.partial(
        pl.kernel,
        out_type=jax.ShapeDtypeStruct((NC, ACCR, D), jnp.float32),
        mesh=_sc_mesh(),
        compiler_params=pltpu.CompilerParams(use_tc_tiling_on_sc=False),
        scratch_types=[
            pltpu.VMEM((KT, CHUNK), jnp.int32),
            pltpu.VMEM((CHUNK, D), jnp.float32),
            pltpu.VMEM_SHARED((ACCR, D), jnp.float32),
        ],
    )
    def k(dst_hbm, ones_hbm, zeros_hbm, out_hbm, didx, buf, acc):
        c = lax.axis_index("c")
        s = lax.axis_index("s")
        lo = c * HALF
        pltpu.sync_copy(dst_hbm.at[pl.ds(s * KT, KT)], didx)
        pltpu.sync_copy(ones_hbm, buf)
        pltpu.sync_copy(zeros_hbm.at[pl.ds(s * RPTA, RPTA)],
                        acc.at[pl.ds(s * RPTA, RPTA)])

        def fix(j, carry):
            for v in range(CHUNK // 16):
                d = didx[j, pl.ds(v * 16, 16)]
                rel = d - lo
                ok = (rel >= 0) & (rel < HALF)
                didx[j, pl.ds(v * 16, 16)] = jnp.where(ok, rel, DUMMY)
            return carry

        lax.fori_loop(0, KT, fix, 0)
        plsc.subcore_barrier()

        def body(j, carry):
            pltpu.sync_copy(buf, acc.at[didx.at[j]], add=True)
            return carry

        lax.fori_loop(0, KT, body, 0)
        plsc.subcore_barrier()
        pltpu.sync_copy(acc.at[pl.ds(s * RPTA, RPTA)],
                        out_hbm.at[c, pl.ds(s * RPTA, RPTA)])

    return k(dst2d, onesrow, zerosN)


def _sc_pass(z, src2d, dst2d, zerosN):
    @functools.partial(
        pl.kernel,
        out_type=jax.ShapeDtypeStruct((NC, ACCR, D), jnp.float32),
        mesh=_sc_mesh(),
        compiler_params=pltpu.CompilerParams(use_tc_tiling_on_sc=False),
        scratch_types=[
            pltpu.VMEM((SEG, CHUNK), jnp.int32),
            pltpu.VMEM((SEG, CHUNK), jnp.int32),
            pltpu.VMEM((NBUF, CHUNK, D), jnp.float32),
            [pltpu.SemaphoreType.DMA] * NBUF,
            pltpu.VMEM_SHARED((ACCR, D), jnp.float32),
        ],
    )
    def k(z_hbm, src_hbm, dst_hbm, zeros_hbm, out_hbm,
          sidx, didx, bufs, sems, acc):
        c = lax.axis_index("c")
        s = lax.axis_index("s")
        lo = c * HALF
        pltpu.sync_copy(zeros_hbm.at[pl.ds(s * RPTA, RPTA)],
                        acc.at[pl.ds(s * RPTA, RPTA)])
        plsc.subcore_barrier()

        for seg in range(NSEG):
            base = s * KT + seg * SEG
            pltpu.sync_copy(src_hbm.at[pl.ds(base, SEG)], sidx)
            pltpu.sync_copy(dst_hbm.at[pl.ds(base, SEG)], didx)

            def fix(j, carry):
                for v in range(CHUNK // 16):
                    d = didx[j, pl.ds(v * 16, 16)]
                    rel = d - lo
                    ok = (rel >= 0) & (rel < HALF)
                    didx[j, pl.ds(v * 16, 16)] = jnp.where(ok, rel, DUMMY)
                    sv = sidx[j, pl.ds(v * 16, 16)]
                    sidx[j, pl.ds(v * 16, 16)] = jnp.where(ok, sv, 0)
                return carry

            lax.fori_loop(0, SEG, fix, 0)

            for kslot in range(NBUF):
                pltpu.async_copy(z_hbm.at[sidx.at[kslot]], bufs.at[kslot],
                                 sems[kslot])

            def body(g, carry):
                for kslot in range(NBUF):
                    j = NBUF * g + kslot
                    pltpu.make_async_copy(z_hbm.at[sidx.at[j]],
                                          bufs.at[kslot], sems[kslot]).wait()
                    pltpu.sync_copy(bufs.at[kslot], acc.at[didx.at[j]],
                                    add=True)

                    @pl.when(j + NBUF < SEG)
                    def _():
                        pltpu.async_copy(z_hbm.at[sidx.at[j + NBUF]],
                                         bufs.at[kslot], sems[kslot])
                return carry

            lax.fori_loop(0, SEG // NBUF, body, 0)

        plsc.subcore_barrier()
        pltpu.sync_copy(acc.at[pl.ds(s * RPTA, RPTA)],
                        out_hbm.at[c, pl.ds(s * RPTA, RPTA)])

    return k(z, src2d, dst2d, zerosN)


def _deg_dis(degp_ref):
    deg = degp_ref[0, :, 0] + 1.0
    return (1.0 / jnp.sqrt(deg))[:, None]


def _tc_prep(degp, x_pad, W0):
    def body(degp_ref, x_ref, w_ref, z_ref):
        dis = _deg_dis(degp_ref)
        xw = jnp.dot(x_ref[...].astype(jnp.bfloat16),
                     w_ref[...].astype(jnp.bfloat16),
                     preferred_element_type=jnp.float32)
        z_ref[...] = xw * dis

    return pl.pallas_call(
        body,
        grid=(GRID,),
        in_specs=[
            pl.BlockSpec((1, BR, D), lambda i: (i // 5, i % 5, 0)),
            pl.BlockSpec((BR, D), lambda i: (i, 0)),
            pl.BlockSpec((D, D), lambda i: (0, 0)),
        ],
        out_specs=pl.BlockSpec((BR, D), lambda i: (i, 0)),
        out_shape=jax.ShapeDtypeStruct((NPAD, D), jnp.float32),
    )(degp, x_pad, W0)


def _tc_mid(degp, acc, z0, b0, W1):
    def body(degp_ref, a_ref, z_ref, b_ref, w_ref, out_ref):
        dis = _deg_dis(degp_ref)
        h = (a_ref[0] + z_ref[...]) * dis + b_ref[...]
        h = jnp.maximum(h, 0.0)
        hw = jnp.dot(h.astype(jnp.bfloat16),
                     w_ref[...].astype(jnp.bfloat16),
                     preferred_element_type=jnp.float32)
        out_ref[...] = hw * dis

    return pl.pallas_call(
        body,
        grid=(GRID,),
        in_specs=[
            pl.BlockSpec((1, BR, D), lambda i: (i // 5, i % 5, 0)),
            pl.BlockSpec((1, BR, D), lambda i: (i // 5, i % 5, 0)),
            pl.BlockSpec((BR, D), lambda i: (i, 0)),
            pl.BlockSpec((1, D), lambda i: (0, 0)),
            pl.BlockSpec((D, D), lambda i: (0, 0)),
        ],
        out_specs=pl.BlockSpec((BR, D), lambda i: (i, 0)),
        out_shape=jax.ShapeDtypeStruct((NPAD, D), jnp.float32),
    )(degp, acc, z0, b0, W1)


def _tc_final(degp, acc, z1, b1, batch3d, linwT, linb):
    def body(degp_ref, a_ref, z_ref, b_ref, batch_ref, lw_ref, lb_ref,
             out_ref, sums, cnts):
        i = pl.program_id(0)
        dis = _deg_dis(degp_ref)
        h = (a_ref[0] + z_ref[...]) * dis + b_ref[...]
        h = jnp.maximum(h, 0.0)
        bvec = batch_ref[0]
        oh = (lax.broadcasted_iota(jnp.int32, (G, BR), 0) == bvec)
        oh = oh.astype(jnp.float32)

        @pl.when(i == 0)
        def _():
            sums[...] = jnp.zeros_like(sums)
            cnts[...] = jnp.zeros_like(cnts)

        sums[...] += jnp.dot(oh, h,
                             preferred_element_type=jnp.float32,
                             precision=lax.Precision.HIGHEST)
        cnts[...] += jnp.dot(oh, jnp.ones((BR, D), jnp.float32),
                             preferred_element_type=jnp.float32,
                             precision=lax.Precision.HIGHEST)

        @pl.when(i == GRID - 1)
        def _():
            pooled = sums[...] / jnp.maximum(cnts[...], 1.0)
            pb = pooled.astype(jnp.bfloat16).astype(jnp.float32)
            lb16 = lw_ref[...].astype(jnp.bfloat16).astype(jnp.float32)
            res = jnp.sum(pb * lb16, axis=1, keepdims=True)
            out_ref[...] = jnp.broadcast_to(res + lb_ref[...], (G, D))

    return pl.pallas_call(
        body,
        grid=(GRID,),
        in_specs=[
            pl.BlockSpec((1, BR, D), lambda i: (i // 5, i % 5, 0)),
            pl.BlockSpec((1, BR, D), lambda i: (i // 5, i % 5, 0)),
            pl.BlockSpec((BR, D), lambda i: (i, 0)),
            pl.BlockSpec((1, D), lambda i: (0, 0)),
            pl.BlockSpec((1, 1, BR), lambda i: (i, 0, 0)),
            pl.BlockSpec((1, D), lambda i: (0, 0)),
            pl.BlockSpec((1, 1), lambda i: (0, 0)),
        ],
        out_specs=pl.BlockSpec((G, D), lambda i: (0, 0)),
        out_shape=jax.ShapeDtypeStruct((G, D), jnp.float32),
        scratch_shapes=[
            pltpu.VMEM((G, D), jnp.float32),
            pltpu.VMEM((G, D), jnp.float32),
        ],
    )(degp, acc, z1, b1, batch3d, linwT, linb)


@jax.jit
def kernel(x, edge_index, batch, W0, b0, W1, b1, lin_W, lin_b):
    src = edge_index[0]
    dst = edge_index[1]
    epad = EPAD - src.shape[0]
    src2d = jnp.concatenate(
        [src, jnp.zeros((epad,), jnp.int32)]).reshape(NS * KT, CHUNK)
    dst2d = jnp.concatenate(
        [dst, jnp.full((epad,), N, jnp.int32)]).reshape(NS * KT, CHUNK)
    x_pad = jnp.pad(x, ((0, NPAD - N), (0, 0)))
    batch3d = jnp.concatenate(
        [batch, jnp.full((NPAD - N,), G, jnp.int32)]).reshape(GRID, 1, BR)
    zerosN = jnp.zeros((ACCR, D), jnp.float32)
    onesrow = jnp.ones((CHUNK, D), jnp.float32)

    degp = _sc_degree(dst2d, onesrow, zerosN)
    z0 = _tc_prep(degp, x_pad, W0)
    acc1 = _sc_pass(z0, src2d, dst2d, zerosN)
    z1 = _tc_mid(degp, acc1, z0, b0.reshape(1, D), W1)
    acc2 = _sc_pass(z1, src2d, dst2d, zerosN)
    out2d = _tc_final(degp, acc2, z1, b1.reshape(1, D), batch3d,
                      lin_W.reshape(1, D), lin_b.reshape(1, 1))
    return out2d[:, 0]


# speedup vs baseline: 13.8391x; 13.8391x over previous
import functools

import jax
import jax.numpy as jnp
from jax import lax
from jax.experimental import pallas as pl
from jax.experimental.pallas import tpu as pltpu
from jax.experimental.pallas import tpu_sc as plsc

N = 10000
D = 128
G = 64
NC = 2
NS = 16
NW = NC * NS
CHUNK = 128
KT = 160
EPAD = NS * KT * CHUNK
NBUF = 4
NSEG = 4
SEG = KT // NSEG
NPAD = 10240
HALF = NPAD // 2
ACCR = 5248
DUMMY = HALF
RPTA = ACCR // NS
BR = 1024
GRID = NPAD // BR


def _sc_mesh():
    return plsc.VectorSubcoreMesh(core_axis_name="c", subcore_axis_name="s")


def _sc_degree(dst2d, onesrow, zerosN):
    @functools.partial(
        pl.kernel,
        out_type=jax.ShapeDtypeStruct((NC, ACCR, D), jnp.float32),
        mesh=_sc_mesh(),
        compiler_params=pltpu.CompilerParams(use_tc_tiling_on_sc=False),
        scratch_types=[
            pltpu.VMEM((KT, CHUNK), jnp.int32),
            pltpu.VMEM((CHUNK, D), jnp.float32),
            pltpu.VMEM_SHARED((ACCR, D), jnp.float32),
        ],
    )
    def k(dst_hbm, ones_hbm, zeros_hbm, out_hbm, didx, buf, acc):
        c = lax.axis_index("c")
        s = lax.axis_index("s")
        lo = c * HALF
        pltpu.sync_copy(dst_hbm.at[pl.ds(s * KT, KT)], didx)
        pltpu.sync_copy(ones_hbm, buf)
        pltpu.sync_copy(zeros_hbm.at[pl.ds(s * RPTA, RPTA)],
                        acc.at[pl.ds(s * RPTA, RPTA)])

        def fix(j, carry):
            for v in range(CHUNK // 16):
                d = didx[j, pl.ds(v * 16, 16)]
                rel = d - lo
                ok = (rel >= 0) & (rel < HALF)
                didx[j, pl.ds(v * 16, 16)] = jnp.where(ok, rel, DUMMY)
            return carry

        lax.fori_loop(0, KT, fix, 0)
        plsc.subcore_barrier()

        def body(j, carry):
            pltpu.sync_copy(buf, acc.at[didx.at[j]], add=True)
            return carry

        lax.fori_loop(0, KT, body, 0)
        plsc.subcore_barrier()
        pltpu.sync_copy(acc.at[pl.ds(s * RPTA, RPTA)],
                        out_hbm.at[c, pl.ds(s * RPTA, RPTA)])

    return k(dst2d, onesrow, zerosN)


def _sc_pass(z, src2d, dst2d, zerosN):
    @functools.partial(
        pl.kernel,
        out_type=jax.ShapeDtypeStruct((NC, ACCR, D), jnp.float32),
        mesh=_sc_mesh(),
        compiler_params=pltpu.CompilerParams(use_tc_tiling_on_sc=False),
        scratch_types=[
            pltpu.VMEM((SEG, CHUNK), jnp.int32),
            pltpu.VMEM((SEG, CHUNK), jnp.int32),
            pltpu.VMEM((NBUF, CHUNK, D), jnp.float32),
            [pltpu.SemaphoreType.DMA] * NBUF,
            pltpu.VMEM_SHARED((ACCR, D), jnp.float32),
        ],
    )
    def k(z_hbm, src_hbm, dst_hbm, zeros_hbm, out_hbm,
          sidx, didx, bufs, sems, acc):
        c = lax.axis_index("c")
        s = lax.axis_index("s")
        lo = c * HALF
        pltpu.sync_copy(zeros_hbm.at[pl.ds(s * RPTA, RPTA)],
                        acc.at[pl.ds(s * RPTA, RPTA)])
        plsc.subcore_barrier()

        for seg in range(NSEG):
            base = s * KT + seg * SEG
            pltpu.sync_copy(src_hbm.at[pl.ds(base, SEG)], sidx)
            pltpu.sync_copy(dst_hbm.at[pl.ds(base, SEG)], didx)

            def fix(j, carry):
                for v in range(CHUNK // 16):
                    d = didx[j, pl.ds(v * 16, 16)]
                    rel = d - lo
                    ok = (rel >= 0) & (rel < HALF)
                    didx[j, pl.ds(v * 16, 16)] = jnp.where(ok, rel, DUMMY)
                return carry

            lax.fori_loop(0, SEG, fix, 0)

            for kslot in range(NBUF):
                pltpu.async_copy(z_hbm.at[sidx.at[kslot]], bufs.at[kslot],
                                 sems[kslot])

            def body(g, carry):
                for kslot in range(NBUF):
                    j = NBUF * g + kslot
                    pltpu.make_async_copy(z_hbm.at[sidx.at[j]],
                                          bufs.at[kslot], sems[kslot]).wait()
                    pltpu.sync_copy(bufs.at[kslot], acc.at[didx.at[j]],
                                    add=True)

                    @pl.when(j + NBUF < SEG)
                    def _():
                        pltpu.async_copy(z_hbm.at[sidx.at[j + NBUF]],
                                         bufs.at[kslot], sems[kslot])
                return carry

            lax.fori_loop(0, SEG // NBUF, body, 0)

        plsc.subcore_barrier()
        pltpu.sync_copy(acc.at[pl.ds(s * RPTA, RPTA)],
                        out_hbm.at[c, pl.ds(s * RPTA, RPTA)])

    return k(z, src2d, dst2d, zerosN)


def _deg_dis(degp_ref):
    deg = degp_ref[0, :, 0] + 1.0
    return (1.0 / jnp.sqrt(deg))[:, None]


def _tc_prep(degp, x_pad, W0):
    def body(degp_ref, x_ref, w_ref, z_ref):
        dis = _deg_dis(degp_ref)
        xw = jnp.dot(x_ref[...].astype(jnp.bfloat16),
                     w_ref[...].astype(jnp.bfloat16),
                     preferred_element_type=jnp.float32)
        z_ref[...] = xw * dis

    return pl.pallas_call(
        body,
        grid=(GRID,),
        in_specs=[
            pl.BlockSpec((1, BR, D), lambda i: (i // 5, i % 5, 0)),
            pl.BlockSpec((BR, D), lambda i: (i, 0)),
            pl.BlockSpec((D, D), lambda i: (0, 0)),
        ],
        out_specs=pl.BlockSpec((BR, D), lambda i: (i, 0)),
        out_shape=jax.ShapeDtypeStruct((NPAD, D), jnp.float32),
    )(degp, x_pad, W0)


def _tc_mid(degp, acc, z0, b0, W1):
    def body(degp_ref, a_ref, z_ref, b_ref, w_ref, out_ref):
        dis = _deg_dis(degp_ref)
        h = (a_ref[0] + z_ref[...]) * dis + b_ref[...]
        h = jnp.maximum(h, 0.0)
        hw = jnp.dot(h.astype(jnp.bfloat16),
                     w_ref[...].astype(jnp.bfloat16),
                     preferred_element_type=jnp.float32)
        out_ref[...] = hw * dis

    return pl.pallas_call(
        body,
        grid=(GRID,),
        in_specs=[
            pl.BlockSpec((1, BR, D), lambda i: (i // 5, i % 5, 0)),
            pl.BlockSpec((1, BR, D), lambda i: (i // 5, i % 5, 0)),
            pl.BlockSpec((BR, D), lambda i: (i, 0)),
            pl.BlockSpec((1, D), lambda i: (0, 0)),
            pl.BlockSpec((D, D), lambda i: (0, 0)),
        ],
        out_specs=pl.BlockSpec((BR, D), lambda i: (i, 0)),
        out_shape=jax.ShapeDtypeStruct((NPAD, D), jnp.float32),
    )(degp, acc, z0, b0, W1)


def _tc_final(degp, acc, z1, b1, batch3d, linwT, linb):
    def body(degp_ref, a_ref, z_ref, b_ref, batch_ref, lw_ref, lb_ref,
             out_ref, sums, cnts):
        i = pl.program_id(0)
        dis = _deg_dis(degp_ref)
        h = (a_ref[0] + z_ref[...]) * dis + b_ref[...]
        h = jnp.maximum(h, 0.0)
        bvec = batch_ref[0]
        oh = (lax.broadcasted_iota(jnp.int32, (G, BR), 0) == bvec)
        oh = oh.astype(jnp.float32)

        @pl.when(i == 0)
        def _():
            sums[...] = jnp.zeros_like(sums)
            cnts[...] = jnp.zeros_like(cnts)

        sums[...] += jnp.dot(oh, h,
                             preferred_element_type=jnp.float32,
                             precision=lax.Precision.HIGHEST)
        cnts[...] += jnp.dot(oh, jnp.ones((BR, D), jnp.float32),
                             preferred_element_type=jnp.float32,
                             precision=lax.Precision.HIGHEST)

        @pl.when(i == GRID - 1)
        def _():
            pooled = sums[...] / jnp.maximum(cnts[...], 1.0)
            pb = pooled.astype(jnp.bfloat16).astype(jnp.float32)
            lb16 = lw_ref[...].astype(jnp.bfloat16).astype(jnp.float32)
            res = jnp.sum(pb * lb16, axis=1, keepdims=True)
            out_ref[...] = jnp.broadcast_to(res + lb_ref[...], (G, D))

    return pl.pallas_call(
        body,
        grid=(GRID,),
        in_specs=[
            pl.BlockSpec((1, BR, D), lambda i: (i // 5, i % 5, 0)),
            pl.BlockSpec((1, BR, D), lambda i: (i // 5, i % 5, 0)),
            pl.BlockSpec((BR, D), lambda i: (i, 0)),
            pl.BlockSpec((1, D), lambda i: (0, 0)),
            pl.BlockSpec((1, 1, BR), lambda i: (i, 0, 0)),
            pl.BlockSpec((1, D), lambda i: (0, 0)),
            pl.BlockSpec((1, 1), lambda i: (0, 0)),
        ],
        out_specs=pl.BlockSpec((G, D), lambda i: (0, 0)),
        out_shape=jax.ShapeDtypeStruct((G, D), jnp.float32),
        scratch_shapes=[
            pltpu.VMEM((G, D), jnp.float32),
            pltpu.VMEM((G, D), jnp.float32),
        ],
    )(degp, acc, z1, b1, batch3d, linwT, linb)


@jax.jit
def kernel(x, edge_index, batch, W0, b0, W1, b1, lin_W, lin_b):
    src = edge_index[0]
    dst = edge_index[1]
    epad = EPAD - src.shape[0]
    src2d = jnp.concatenate(
        [src, jnp.zeros((epad,), jnp.int32)]).reshape(NS * KT, CHUNK)
    dst2d = jnp.concatenate(
        [dst, jnp.full((epad,), N, jnp.int32)]).reshape(NS * KT, CHUNK)
    x_pad = jnp.pad(x, ((0, NPAD - N), (0, 0)))
    batch3d = jnp.concatenate(
        [batch, jnp.full((NPAD - N,), G, jnp.int32)]).reshape(GRID, 1, BR)
    zerosN = jnp.zeros((ACCR, D), jnp.float32)
    onesrow = jnp.ones((CHUNK, D), jnp.float32)

    degp = _sc_degree(dst2d, onesrow, zerosN)
    z0 = _tc_prep(degp, x_pad, W0)
    acc1 = _sc_pass(z0, src2d, dst2d, zerosN)
    z1 = _tc_mid(degp, acc1, z0, b0.reshape(1, D), W1)
    acc2 = _sc_pass(z1, src2d, dst2d, zerosN)
    out2d = _tc_final(degp, acc2, z1, b1.reshape(1, D), batch3d,
                      lin_W.reshape(1, D), lin_b.reshape(1, 1))
    return out2d[:, 0]
